# baseline (speedup 1.0000x reference)
import functools

import jax
import jax.numpy as jnp
from jax.experimental import pallas as pl
from jax.experimental.pallas import tpu as pltpu


def _gnn_body(adj_ref, w_ref, f_ref, out_ref, p_ref):
    @pl.when(pl.program_id(1) == 0)
    def _():
        p_ref[...] = jnp.dot(adj_ref[...], w_ref[...],
                             preferred_element_type=jnp.float32)

    o = jnp.dot(p_ref[...], f_ref[...], preferred_element_type=jnp.float32)
    out_ref[...] = jnp.maximum(o, 0.0)


@functools.partial(jax.jit, static_argnames=("block_m", "block_n"))
def _gnn(features, adj, weight, block_m=512, block_n=2048):
    n, in_f = adj.shape
    out_f = features.shape[0]
    n_out = features.shape[1]
    grid = (n // block_m, n_out // block_n)
    return pl.pallas_call(
        _gnn_body,
        grid=grid,
        in_specs=[
            pl.BlockSpec((block_m, in_f), lambda i, j: (i, 0)),
            pl.BlockSpec((in_f, out_f), lambda i, j: (0, 0)),
            pl.BlockSpec((out_f, block_n), lambda i, j: (0, j)),
        ],
        out_specs=pl.BlockSpec((block_m, block_n), lambda i, j: (i, j)),
        out_shape=jax.ShapeDtypeStruct((n, n_out), jnp.float32),
        scratch_shapes=[pltpu.VMEM((block_m, out_f), jnp.float32)],
        compiler_params=pltpu.CompilerParams(
            dimension_semantics=("parallel", "arbitrary"),
        ),
    )(adj, weight, features)


def kernel(features, adj, weight):
    return _gnn(features, adj, weight)


# speedup vs baseline: 1.3967x; 1.3967x over previous
import functools

import jax
import jax.numpy as jnp
from jax.experimental import pallas as pl
from jax.experimental.pallas import tpu as pltpu

_PREC = jax.lax.Precision.DEFAULT


def _gnn_body(adj_ref, w_ref, f_ref, out_ref):
    p = jnp.dot(adj_ref[...], w_ref[...],
                preferred_element_type=jnp.float32, precision=_PREC)
    o = jnp.dot(p, f_ref[...],
                preferred_element_type=jnp.float32, precision=_PREC)
    out_ref[...] = jnp.maximum(o, 0.0)


@functools.partial(jax.jit, static_argnames=("block_m",))
def _gnn(features, adj, weight, block_m=512):
    n, in_f = adj.shape
    out_f = features.shape[0]
    n_out = features.shape[1]
    grid = (n // block_m,)
    return pl.pallas_call(
        _gnn_body,
        grid=grid,
        in_specs=[
            pl.BlockSpec((block_m, in_f), lambda i: (i, 0)),
            pl.BlockSpec((in_f, out_f), lambda i: (0, 0)),
            pl.BlockSpec((out_f, n_out), lambda i: (0, 0)),
        ],
        out_specs=pl.BlockSpec((block_m, n_out), lambda i: (i, 0)),
        out_shape=jax.ShapeDtypeStruct((n, n_out), jnp.float32),
        compiler_params=pltpu.CompilerParams(
            dimension_semantics=("parallel",),
        ),
    )(adj, weight, features)


def kernel(features, adj, weight):
    return _gnn(features, adj, weight)
